# baseline (speedup 1.0000x reference)
import functools

import jax
import jax.numpy as jnp
from jax import lax
from jax.experimental import pallas as pl
from jax.experimental.pallas import tpu as pltpu

_T = 2048
_H = 1024
_E = 8
_K = 2
_DFF = 1024
_NG = 4
_TG = 2

_NEG = -1e30


def _min_index(mask, idx):
    return jnp.min(jnp.where(mask, idx, 10**9), axis=1, keepdims=True)


def _routing_body(scores_ref, bias_ref, comb_ref):
    scores = scores_ref[...]
    s_choice = scores + bias_ref[...]

    iota_e = lax.broadcasted_iota(jnp.int32, (_T, _E), 1)

    left = jnp.concatenate([s_choice[:, 1:], s_choice[:, :1]], axis=1)
    right = jnp.concatenate([s_choice[:, -1:], s_choice[:, :-1]], axis=1)
    partner = jnp.where(iota_e % 2 == 0, left, right)
    gsum = s_choice + partner

    m1 = jnp.max(gsum, axis=1, keepdims=True)
    a1 = _min_index(gsum == m1, iota_e) // 2
    gs2 = jnp.where(iota_e // 2 == a1, _NEG, gsum)
    m2 = jnp.max(gs2, axis=1, keepdims=True)
    a2 = _min_index(gs2 == m2, iota_e) // 2
    grp_ok = (iota_e // 2 == a1) | (iota_e // 2 == a2)

    masked = jnp.where(grp_ok, s_choice, _NEG)
    e_m1 = jnp.max(masked, axis=1, keepdims=True)
    e_i1 = _min_index(masked == e_m1, iota_e)
    masked2 = jnp.where(iota_e == e_i1, _NEG, masked)
    e_m2 = jnp.max(masked2, axis=1, keepdims=True)
    e_i2 = _min_index(masked2 == e_m2, iota_e)

    w1 = jnp.sum(jnp.where(iota_e == e_i1, scores, 0.0), axis=1, keepdims=True)
    w2 = jnp.sum(jnp.where(iota_e == e_i2, scores, 0.0), axis=1, keepdims=True)
    wsum = w1 + w2 + 1e-20
    comb_ref[...] = (jnp.where(iota_e == e_i1, w1 / wsum, 0.0)
                     + jnp.where(iota_e == e_i2, w2 / wsum, 0.0))


def _routing(hs, gw, bias):
    scores = jax.nn.sigmoid((hs @ gw.T).astype(jnp.float32))
    return pl.pallas_call(
        _routing_body,
        out_shape=jax.ShapeDtypeStruct((_T, _E), jnp.float32),
    )(scores, bias.reshape(1, _E))


_TM = 128
_NT = _T // _TM


def _silu_mul(gu):
    g = gu[:, :_DFF]
    u = gu[:, _DFF:]
    return (g * jax.nn.sigmoid(g)) * u


def _expert_body(hs_ref, wgu_ref, wd_ref, swgu_ref, swd_ref, comb_ref, out_ref):
    e = pl.program_id(0)
    t = pl.program_id(1)
    x = hs_ref[...]
    gu = lax.dot_general(x, wgu_ref[0], (((1,), (1,)), ((), ())),
                         preferred_element_type=jnp.float32)
    act = _silu_mul(gu)
    eo = lax.dot_general(act, wd_ref[0], (((1,), (1,)), ((), ())),
                         preferred_element_type=jnp.float32)
    iota_e = lax.broadcasted_iota(jnp.int32, (_TM, _E), 1)
    c = jnp.sum(jnp.where(iota_e == e, comb_ref[...], 0.0), axis=1,
                keepdims=True)
    contrib = eo * c

    rows = pl.ds(t * _TM, _TM)

    @pl.when(e == 0)
    def _init():
        sgu = lax.dot_general(x, swgu_ref[...], (((1,), (1,)), ((), ())),
                              preferred_element_type=jnp.float32)
        sout = lax.dot_general(_silu_mul(sgu), swd_ref[...],
                               (((1,), (1,)), ((), ())),
                               preferred_element_type=jnp.float32)
        out_ref[rows, :] = sout + contrib

    @pl.when(e > 0)
    def _acc():
        out_ref[rows, :] = out_ref[rows, :] + contrib


def _experts(hs, wgu, wd, swgu, swd, comb):
    return pl.pallas_call(
        _expert_body,
        grid=(_E, _NT),
        in_specs=[
            pl.BlockSpec((_TM, _H), lambda e, t: (t, 0)),
            pl.BlockSpec((1, 2 * _DFF, _H), lambda e, t: (e, 0, 0)),
            pl.BlockSpec((1, _H, _DFF), lambda e, t: (e, 0, 0)),
            pl.BlockSpec((2 * _DFF, _H), lambda e, t: (0, 0)),
            pl.BlockSpec((_H, _DFF), lambda e, t: (0, 0)),
            pl.BlockSpec((_TM, _E), lambda e, t: (t, 0)),
        ],
        out_specs=pl.BlockSpec((_T, _H), lambda e, t: (0, 0)),
        out_shape=jax.ShapeDtypeStruct((_T, _H), jnp.float32),
    )(hs, wgu, wd, swgu, swd, comb)


def kernel(hidden_states, gate_weight, e_score_correction_bias, w_gate_up,
           w_down, shared_w_gate_up, shared_w_down):
    comb = _routing(hidden_states, gate_weight, e_score_correction_bias)
    return _experts(hidden_states, w_gate_up, w_down, shared_w_gate_up,
                    shared_w_down, comb)
